# baseline (speedup 1.0000x reference)
import functools

import jax
import jax.numpy as jnp
from jax import lax
from jax.experimental import pallas as pl
from jax.experimental.pallas import tpu as pltpu
from jax.experimental.pallas import tpu_sc as plsc

_NE, _D = 10000, 128
_NREG = 5000
_NMP = 8000
_EF, _EMP = 320000, 160000
_NC, _NS, _L = 2, 16, 16
_NW = _NC * _NS
_NEP = 10240
_NMPP = 8192
_CF, _CFW = 80, 128
_CH, _CHW = 80, 128
_F32 = jnp.float32


def _mesh():
    return plsc.VectorSubcoreMesh(core_axis_name="c", subcore_axis_name="s")


def _fill(ref, n16, val):
    v = jnp.full((_L,), val, _F32)

    def body(i, _):
        ref[pl.ds(i * _L, _L)] = v
        return 0

    lax.fori_loop(0, n16, body, 0)


def _zero_rows(rows, n):
    z = jnp.zeros((_L,), _F32)

    def body(i, _):
        for j in range(8):
            rows[i, pl.ds(j * _L, _L)] = z
        return 0

    lax.fori_loop(0, n, body, 0)


_CB = 10


def _make_sc_agg(n_rows, chunks, cw, with_counts):
    rpt = n_rows // _NS
    assert rpt % 128 == 0

    def body(src_hbm, dst_hbm, y_hbm, *rest):
        if with_counts:
            (outp, outc, sidx, didx, rows, ones, zc,
             acc, cacc, gsem, isem) = rest
        else:
            (outp, sidx, didx, rows, acc, gsem, isem) = rest
        cid = lax.axis_index("c")
        sid = lax.axis_index("s")
        wid = cid * _NS + sid

        _zero_rows(rows, cw)
        if with_counts:
            _fill(ones, cw // _L, 1.0)
            _fill(zc, rpt // _L, 0.0)

        def zacc(k, _):
            pltpu.sync_copy(rows, acc.at[pl.ds(sid * rpt + k * cw, cw)])
            return 0

        lax.fori_loop(0, rpt // cw, zacc, 0)
        if with_counts:
            pltpu.sync_copy(zc, cacc.at[pl.ds(sid * rpt, rpt)])
        plsc.subcore_barrier()

        pltpu.sync_copy(src_hbm.at[wid], sidx)
        pltpu.sync_copy(dst_hbm.at[wid], didx)

        def step(j, _):
            pltpu.async_copy(y_hbm.at[sidx.at[j]], rows, gsem).wait()
            pltpu.sync_copy(rows, acc.at[didx.at[j]], add=True)
            if with_counts:
                pltpu.sync_copy(ones, cacc.at[didx.at[j]], add=True)
            return 0

        lax.fori_loop(0, chunks, step, 0)
        plsc.subcore_barrier()

        def wout(k, _):
            r = sid * rpt + k * 128
            pltpu.sync_copy(acc.at[pl.ds(r, 128)],
                            outp.at[pl.ds(cid * n_rows + r, 128)])
            return 0

        lax.fori_loop(0, rpt // 128, wout, 0)
        if with_counts:
            pltpu.sync_copy(cacc.at[pl.ds(sid * rpt, rpt)],
                            outc.at[pl.ds(cid * n_rows + sid * rpt, rpt)])

    out_type = [jax.ShapeDtypeStruct((_NC * n_rows, _D), _F32)]
    scratch = [
        pltpu.VMEM((chunks, cw), jnp.int32),
        pltpu.VMEM((chunks, cw), jnp.int32),
        pltpu.VMEM((cw, _D), _F32),
    ]
    if with_counts:
        out_type.append(jax.ShapeDtypeStruct((_NC * n_rows,), _F32))
        scratch += [pltpu.VMEM((cw,), _F32), pltpu.VMEM((rpt,), _F32)]
    scratch.append(pltpu.VMEM_SHARED((n_rows, _D), _F32))
    if with_counts:
        scratch.append(pltpu.VMEM_SHARED((n_rows,), _F32))
    scratch += [pltpu.SemaphoreType.DMA] * 2
    return pl.kernel(
        body,
        out_type=out_type if with_counts else out_type[0],
        mesh=_mesh(),
        scratch_types=scratch,
    )


def _make_sc_gather(n_out, table_rows):
    kpw = n_out // _NW // 128

    def body(table_hbm, idx_hbm, out_hbm, idxv, rows, sem):
        cid = lax.axis_index("c")
        sid = lax.axis_index("s")
        wid = cid * _NS + sid
        pltpu.sync_copy(idx_hbm.at[wid], idxv)
        for k in range(kpw):
            pltpu.async_copy(table_hbm.at[idxv.at[k]], rows, sem).wait()
            pltpu.sync_copy(rows, out_hbm.at[pl.ds(wid * kpw * 128 + k * 128, 128)])

    return pl.kernel(
        body,
        out_type=jax.ShapeDtypeStruct((n_out, _D), _F32),
        mesh=_mesh(),
        scratch_types=[
            pltpu.VMEM((kpw, 128), jnp.int32),
            pltpu.VMEM((128, _D), _F32),
            pltpu.SemaphoreType.DMA,
        ],
    )


_BLK = 1024


def _row_spec(blk=_BLK):
    return pl.BlockSpec((blk, _D), lambda i: (i, 0))


def _w_spec():
    return pl.BlockSpec((_D, _D), lambda i: (0, 0))


def _b_spec():
    return pl.BlockSpec((1, _D), lambda i: (0, 0))


def _tc_head(ep, wm, bm, wrel):
    def body(ep_r, wm_r, bm_r, wr_r, e_out, y_out):
        e = jnp.dot(ep_r[...], wm_r[...], preferred_element_type=_F32) + bm_r[...]
        e_out[...] = e
        y_out[...] = jnp.dot(e, wr_r[...], preferred_element_type=_F32)

    n = ep.shape[0]
    return pl.pallas_call(
        body,
        grid=(n // _BLK,),
        in_specs=[_row_spec(), _w_spec(), _b_spec(), _w_spec()],
        out_specs=[_row_spec(), _row_spec()],
        out_shape=[jax.ShapeDtypeStruct((n, _D), _F32)] * 2,
    )(ep, wm, bm, wrel)


def _tc_combine_full(p0, p1, c0, c1, e, wroot, b, wrel_next):
    with_next = wrel_next is not None

    def body(p0_r, p1_r, c0_r, c1_r, e_r, wr_r, b_r, *rest):
        if with_next:
            wn_r, e_out, y_out = rest
        else:
            (e_out,) = rest
        cnt = jnp.maximum(c0_r[...] + c1_r[...], 1.0)
        agg = (p0_r[...] + p1_r[...]) / cnt[:, None]
        enew = jnp.tanh(agg + jnp.dot(e_r[...], wr_r[...],
                                      preferred_element_type=_F32) + b_r[...])
        e_out[...] = enew
        if with_next:
            y_out[...] = jnp.dot(enew, wn_r[...], preferred_element_type=_F32)

    n = e.shape[0]
    c_spec = pl.BlockSpec((_BLK,), lambda i: (i,))
    in_specs = [_row_spec(), _row_spec(), c_spec, c_spec, _row_spec(),
                _w_spec(), _b_spec()]
    args = [p0, p1, c0, c1, e, wroot, b]
    out_specs = [_row_spec()]
    out_shape = [jax.ShapeDtypeStruct((n, _D), _F32)]
    if with_next:
        in_specs.append(_w_spec())
        args.append(wrel_next)
        out_specs.append(_row_spec())
        out_shape.append(jax.ShapeDtypeStruct((n, _D), _F32))
    res = pl.pallas_call(
        body, grid=(n // _BLK,), in_specs=in_specs, out_specs=out_specs,
        out_shape=out_shape,
    )(*args)
    return res if with_next else res[0]


def _tc_mm_per_graph(x, w):
    gblk = _NMPP // _BLK

    def body(x_r, w_r, y_out):
        y_out[...] = jnp.dot(x_r[...], w_r[0], preferred_element_type=_F32)

    n = x.shape[0]
    return pl.pallas_call(
        body,
        grid=(n // _BLK,),
        in_specs=[_row_spec(),
                  pl.BlockSpec((1, _D, _D), lambda i: (i // gblk, 0, 0))],
        out_specs=_row_spec(),
        out_shape=jax.ShapeDtypeStruct((n, _D), _F32),
    )(x, w)


def _tc_combine_han(s, c, e, wroot, b, wrel_next):
    with_next = wrel_next is not None
    gblk = _NMPP // _BLK

    def body(s_r, c_r, e_r, wr_r, b_r, *rest):
        if with_next:
            wn_r, e_out, y_out = rest
        else:
            (e_out,) = rest
        cnt = jnp.maximum(c_r[...], 1.0)
        agg = s_r[...] / cnt[:, None]
        enew = jax.nn.relu(agg + jnp.dot(e_r[...], wr_r[0],
                                         preferred_element_type=_F32) + b_r[0])
        e_out[...] = enew
        if with_next:
            y_out[...] = jnp.dot(enew, wn_r[0], preferred_element_type=_F32)

    n = e.shape[0]
    wg_spec = pl.BlockSpec((1, _D, _D), lambda i: (i // gblk, 0, 0))
    bg_spec = pl.BlockSpec((1, 1, _D), lambda i: (i // gblk, 0, 0))
    c_spec = pl.BlockSpec((_BLK,), lambda i: (i,))
    in_specs = [_row_spec(), c_spec, _row_spec(), wg_spec, bg_spec]
    args = [s, c, e, wroot, b]
    out_specs = [_row_spec()]
    out_shape = [jax.ShapeDtypeStruct((n, _D), _F32)]
    if with_next:
        in_specs.append(wg_spec)
        args.append(wrel_next)
        out_specs.append(_row_spec())
        out_shape.append(jax.ShapeDtypeStruct((n, _D), _F32))
    res = pl.pallas_call(
        body, grid=(n // _BLK,), in_specs=in_specs, out_specs=out_specs,
        out_shape=out_shape,
    )(*args)
    return res if with_next else res[0]


def _tc_final(sem0, sem1, e5k, mp, pw1, pb1, pw2, predw, predb, erw, erb):
    def body(s0_r, s1_r, e_r, mp_r, pw1_r, pb1_r, pw2_r, prw_r, prb_r, erw_r,
             erb_r, out):
        t = jnp.tanh(jnp.dot(mp_r[...], pw1_r[...],
                             preferred_element_type=_F32) + pb1_r[...])
        w = jnp.dot(t, pw2_r[...], preferred_element_type=_F32)
        m = jnp.max(w, axis=0, keepdims=True)
        ew = jnp.exp(w - m)
        wn = ew / jnp.sum(ew, axis=0, keepdims=True)
        h = s0_r[...] * wn[0:1, :] + s1_r[...] * wn[1:2, :]
        ereg = jnp.dot(h, prw_r[...], preferred_element_type=_F32) + prb_r[...]
        ereg = ereg + e_r[...]
        pred = jnp.dot(ereg, erw_r[...], preferred_element_type=_F32) + erb_r[...]
        out[...] = jnp.broadcast_to(pred, (pred.shape[0], _D))

    full = lambda shape: pl.BlockSpec(shape, lambda: tuple(0 for _ in shape))
    mpd = mp.shape[1]
    return pl.pallas_call(
        body,
        in_specs=[full((_NREG, _D)), full((_NREG, _D)), full((_NREG, _D)),
                  full((2, mpd)), full((mpd, _D)), full((1, _D)),
                  full((_D, 1)), full((_D, _D)), full((1, _D)),
                  full((_D, 1)), full((1, 1))],
        out_specs=full((_NREG, _D)),
        out_shape=jax.ShapeDtypeStruct((_NREG, _D), _F32),
    )(sem0, sem1, e5k, mp, pw1, pb1, pw2, predw, predb, erw, erb)


def _pad1(x, n, val):
    return jnp.concatenate([x, jnp.full((n - x.shape[0],), val, x.dtype)])


def _pad_spread(x, n, base, room):
    m = n - x.shape[0]
    fill = base + (jnp.arange(m, dtype=x.dtype) % room)
    return jnp.concatenate([x, fill])


def kernel(edge_index, gs0_edge_index, gs0_eids, gs1_edge_index, gs1_eids,
           metapath_emb, task_desc_emb, E_pretrain, Wm, bm,
           rgcn_Wroot, rgcn_Wrel, rgcn_b, han_Wroot, han_Wrel, han_b,
           projW1, projb1, projW2, predW, predb, erW, erb):
    ep = jnp.concatenate(
        [E_pretrain, jnp.zeros((_NEP - _NE, _D), _F32)], axis=0)
    nef = _NW * _CF * _CFW
    srcf = _pad1(edge_index[0], nef, 0).reshape(_NW, _CF, _CFW)
    dstf = _pad_spread(edge_index[1], nef, _NE, _NEP - _NE).reshape(
        _NW, _CF, _CFW)

    neh = _NS * _CH * _CHW
    srch = jnp.concatenate([
        _pad1(gs0_edge_index[0], neh, 0),
        _pad1(gs1_edge_index[0] + _NMPP, neh, _NMPP),
    ]).reshape(_NW, _CH, _CHW)
    dsth = jnp.concatenate([
        _pad_spread(gs0_edge_index[1], neh, _NMP, _NMPP - _NMP),
        _pad_spread(gs1_edge_index[1], neh, _NMP, _NMPP - _NMP),
    ]).reshape(_NW, _CH, _CHW)

    eids = jnp.concatenate([
        _pad1(gs0_eids, _NMPP, 0), _pad1(gs1_eids, _NMPP, 0),
    ]).reshape(_NW, (2 * _NMPP) // _NW // 128, 128)

    bm2 = bm.reshape(1, _D)
    rb = rgcn_b.reshape(2, 1, _D)
    hb = han_b.reshape(4, 1, _D)

    agg_full_c = _make_sc_agg(_NEP, _CF, _CFW, True)
    agg_han_c = _make_sc_agg(_NMPP, _CH, _CHW, True)
    gather = _make_sc_gather(2 * _NMPP, _NEP)

    e0, y0 = _tc_head(ep, Wm, bm2, rgcn_Wrel[0])
    p, c = agg_full_c(srcf, dstf, y0)
    e1, y1 = _tc_combine_full(p[:_NEP], p[_NEP:], c[:_NEP], c[_NEP:],
                              e0, rgcn_Wroot[0], rb[0], rgcn_Wrel[1])
    p, _c2 = agg_full_c(srcf, dstf, y1)
    e2 = _tc_combine_full(p[:_NEP], p[_NEP:], c[:_NEP], c[_NEP:],
                          e1, rgcn_Wroot[1], rb[1], None)

    ef = gather(e2, eids)
    wrel_j0 = jnp.stack([han_Wrel[0], han_Wrel[2]])
    wrel_j1 = jnp.stack([han_Wrel[1], han_Wrel[3]])
    wroot_j0 = jnp.stack([han_Wroot[0], han_Wroot[2]])
    wroot_j1 = jnp.stack([han_Wroot[1], han_Wroot[3]])
    b_j0 = jnp.stack([hb[0], hb[2]])
    b_j1 = jnp.stack([hb[1], hb[3]])

    yh = _tc_mm_per_graph(ef, wrel_j0)
    s, ch = agg_han_c(srch, dsth, yh)
    ef1, yh2 = _tc_combine_han(s, ch, ef, wroot_j0, b_j0, wrel_j1)
    s, _c3 = agg_han_c(srch, dsth, yh2)
    ef2 = _tc_combine_han(s, ch, ef1, wroot_j1, b_j1, None)

    out = _tc_final(ef2[:_NREG], ef2[_NMPP:_NMPP + _NREG], e2[:_NREG],
                    metapath_emb, projW1, projb1.reshape(1, _D), projW2,
                    predW, predb.reshape(1, _D), erW, erb.reshape(1, 1))
    return out[:, :1]


# speedup vs baseline: 1.0048x; 1.0048x over previous
import jax
import jax.numpy as jnp
from jax import lax
from jax.experimental import pallas as pl
from jax.experimental.pallas import tpu as pltpu
from jax.experimental.pallas import tpu_sc as plsc

_NE, _D = 10000, 128
_NREG = 5000
_NMP = 8000
_EF, _EMP = 320000, 160000
_NC, _NS, _L = 2, 16, 16
_NW = _NC * _NS
_NEP = 10240
_NMPP = 8192
_CF = 80
_CH = 80
_F32 = jnp.float32


def _mesh():
    return plsc.VectorSubcoreMesh(core_axis_name="c", subcore_axis_name="s")


def _fill(ref, n16, val):
    v = jnp.full((_L,), val, _F32)

    def body(i, _):
        ref[pl.ds(i * _L, _L)] = v
        return 0

    lax.fori_loop(0, n16, body, 0)


def _zero_rows(rows):
    z = jnp.zeros((_L,), _F32)

    def body(i, _):
        for j in range(8):
            rows[i, pl.ds(j * _L, _L)] = z
        return 0

    lax.fori_loop(0, 128, body, 0)


def _make_sc_agg(n_rows, chunks, rows_out):
    rpt = n_rows // _NS
    assert rpt % 128 == 0

    def body(src_hbm, dst_hbm, y_hbm, outp, outc, sidx, didx, rows, ones, zc,
             acc, cacc, sem):
        cid = lax.axis_index("c")
        sid = lax.axis_index("s")
        wid = cid * _NS + sid

        _zero_rows(rows)
        _fill(ones, 128 // _L, 1.0)
        _fill(zc, rpt // _L, 0.0)

        def zacc(k, _):
            pltpu.sync_copy(rows, acc.at[pl.ds(sid * rpt + k * 128, 128)])
            return 0

        lax.fori_loop(0, rpt // 128, zacc, 0)
        pltpu.sync_copy(zc, cacc.at[pl.ds(sid * rpt, rpt)])
        plsc.subcore_barrier()

        pltpu.sync_copy(src_hbm.at[wid], sidx)
        pltpu.sync_copy(dst_hbm.at[wid], didx)

        def step(j, _):
            pltpu.async_copy(y_hbm.at[sidx.at[j]], rows, sem).wait()
            pltpu.sync_copy(rows, acc.at[didx.at[j]], add=True)
            pltpu.sync_copy(ones, cacc.at[didx.at[j]], add=True)
            return 0

        lax.fori_loop(0, chunks, step, 0)
        plsc.subcore_barrier()

        def wout(k, _):
            r = sid * rpt + k * 128
            pltpu.sync_copy(acc.at[pl.ds(r, 128)],
                            outp.at[pl.ds(cid * n_rows + r, 128)])
            return 0

        lax.fori_loop(0, rpt // 128, wout, 0)
        pltpu.sync_copy(cacc.at[pl.ds(sid * rpt, rpt)],
                        outc.at[pl.ds(cid * n_rows + sid * rpt, rpt)])

    return pl.kernel(
        body,
        out_type=[
            jax.ShapeDtypeStruct((_NC * n_rows, _D), _F32),
            jax.ShapeDtypeStruct((_NC * n_rows,), _F32),
        ],
        mesh=_mesh(),
        scratch_types=[
            pltpu.VMEM((chunks, 128), jnp.int32),
            pltpu.VMEM((chunks, 128), jnp.int32),
            pltpu.VMEM((128, _D), _F32),
            pltpu.VMEM((128,), _F32),
            pltpu.VMEM((rpt,), _F32),
            pltpu.VMEM_SHARED((n_rows, _D), _F32),
            pltpu.VMEM_SHARED((n_rows,), _F32),
            pltpu.SemaphoreType.DMA,
        ],
    )


def _make_sc_gather(n_out, table_rows):
    kpw = n_out // _NW // 128

    def body(table_hbm, idx_hbm, out_hbm, idxv, rows, sem):
        cid = lax.axis_index("c")
        sid = lax.axis_index("s")
        wid = cid * _NS + sid
        pltpu.sync_copy(idx_hbm.at[wid], idxv)
        for k in range(kpw):
            pltpu.async_copy(table_hbm.at[idxv.at[k]], rows, sem).wait()
            pltpu.sync_copy(rows, out_hbm.at[pl.ds(wid * kpw * 128 + k * 128, 128)])

    return pl.kernel(
        body,
        out_type=jax.ShapeDtypeStruct((n_out, _D), _F32),
        mesh=_mesh(),
        scratch_types=[
            pltpu.VMEM((kpw, 128), jnp.int32),
            pltpu.VMEM((128, _D), _F32),
            pltpu.SemaphoreType.DMA,
        ],
    )


_BLK = 1024


def _row_spec(blk=_BLK):
    return pl.BlockSpec((blk, _D), lambda i: (i, 0))


def _w_spec():
    return pl.BlockSpec((_D, _D), lambda i: (0, 0))


def _b_spec():
    return pl.BlockSpec((1, _D), lambda i: (0, 0))


def _tc_head(ep, wm, bm, wrel):
    def body(ep_r, wm_r, bm_r, wr_r, e_out, y_out):
        e = jnp.dot(ep_r[...], wm_r[...], preferred_element_type=_F32) + bm_r[...]
        e_out[...] = e
        y_out[...] = jnp.dot(e, wr_r[...], preferred_element_type=_F32)

    n = ep.shape[0]
    return pl.pallas_call(
        body,
        grid=(n // _BLK,),
        in_specs=[_row_spec(), _w_spec(), _b_spec(), _w_spec()],
        out_specs=[_row_spec(), _row_spec()],
        out_shape=[jax.ShapeDtypeStruct((n, _D), _F32)] * 2,
    )(ep, wm, bm, wrel)


def _tc_combine_full(p0, p1, c0, c1, e, wroot, b, wrel_next):
    with_next = wrel_next is not None

    def body(p0_r, p1_r, c0_r, c1_r, e_r, wr_r, b_r, *rest):
        if with_next:
            wn_r, e_out, y_out = rest
        else:
            (e_out,) = rest
        cnt = jnp.maximum(c0_r[...] + c1_r[...], 1.0)
        agg = (p0_r[...] + p1_r[...]) / cnt[:, None]
        enew = jnp.tanh(agg + jnp.dot(e_r[...], wr_r[...],
                                      preferred_element_type=_F32) + b_r[...])
        e_out[...] = enew
        if with_next:
            y_out[...] = jnp.dot(enew, wn_r[...], preferred_element_type=_F32)

    n = e.shape[0]
    c_spec = pl.BlockSpec((_BLK,), lambda i: (i,))
    in_specs = [_row_spec(), _row_spec(), c_spec, c_spec, _row_spec(),
                _w_spec(), _b_spec()]
    args = [p0, p1, c0, c1, e, wroot, b]
    out_specs = [_row_spec()]
    out_shape = [jax.ShapeDtypeStruct((n, _D), _F32)]
    if with_next:
        in_specs.append(_w_spec())
        args.append(wrel_next)
        out_specs.append(_row_spec())
        out_shape.append(jax.ShapeDtypeStruct((n, _D), _F32))
    res = pl.pallas_call(
        body, grid=(n // _BLK,), in_specs=in_specs, out_specs=out_specs,
        out_shape=out_shape,
    )(*args)
    return res if with_next else res[0]


def _tc_mm_per_graph(x, w):
    gblk = _NMPP // _BLK

    def body(x_r, w_r, y_out):
        y_out[...] = jnp.dot(x_r[...], w_r[0], preferred_element_type=_F32)

    n = x.shape[0]
    return pl.pallas_call(
        body,
        grid=(n // _BLK,),
        in_specs=[_row_spec(),
                  pl.BlockSpec((1, _D, _D), lambda i: (i // gblk, 0, 0))],
        out_specs=_row_spec(),
        out_shape=jax.ShapeDtypeStruct((n, _D), _F32),
    )(x, w)


def _tc_combine_han(s, c, e, wroot, b, wrel_next):
    with_next = wrel_next is not None
    gblk = _NMPP // _BLK

    def body(s_r, c_r, e_r, wr_r, b_r, *rest):
        if with_next:
            wn_r, e_out, y_out = rest
        else:
            (e_out,) = rest
        cnt = jnp.maximum(c_r[...], 1.0)
        agg = s_r[...] / cnt[:, None]
        enew = jax.nn.relu(agg + jnp.dot(e_r[...], wr_r[0],
                                         preferred_element_type=_F32) + b_r[0])
        e_out[...] = enew
        if with_next:
            y_out[...] = jnp.dot(enew, wn_r[0], preferred_element_type=_F32)

    n = e.shape[0]
    wg_spec = pl.BlockSpec((1, _D, _D), lambda i: (i // gblk, 0, 0))
    bg_spec = pl.BlockSpec((1, 1, _D), lambda i: (i // gblk, 0, 0))
    c_spec = pl.BlockSpec((_BLK,), lambda i: (i,))
    in_specs = [_row_spec(), c_spec, _row_spec(), wg_spec, bg_spec]
    args = [s, c, e, wroot, b]
    out_specs = [_row_spec()]
    out_shape = [jax.ShapeDtypeStruct((n, _D), _F32)]
    if with_next:
        in_specs.append(wg_spec)
        args.append(wrel_next)
        out_specs.append(_row_spec())
        out_shape.append(jax.ShapeDtypeStruct((n, _D), _F32))
    res = pl.pallas_call(
        body, grid=(n // _BLK,), in_specs=in_specs, out_specs=out_specs,
        out_shape=out_shape,
    )(*args)
    return res if with_next else res[0]


def _tc_final(sem0, sem1, e5k, mp, pw1, pb1, pw2, predw, predb, erw, erb):
    def body(s0_r, s1_r, e_r, mp_r, pw1_r, pb1_r, pw2_r, prw_r, prb_r, erw_r,
             erb_r, out):
        t = jnp.tanh(jnp.dot(mp_r[...], pw1_r[...],
                             preferred_element_type=_F32) + pb1_r[...])
        w = jnp.dot(t, pw2_r[...], preferred_element_type=_F32)
        m = jnp.max(w, axis=0, keepdims=True)
        ew = jnp.exp(w - m)
        wn = ew / jnp.sum(ew, axis=0, keepdims=True)
        h = s0_r[...] * wn[0:1, :] + s1_r[...] * wn[1:2, :]
        ereg = jnp.dot(h, prw_r[...], preferred_element_type=_F32) + prb_r[...]
        ereg = ereg + e_r[...]
        pred = jnp.dot(ereg, erw_r[...], preferred_element_type=_F32) + erb_r[...]
        out[...] = jnp.broadcast_to(pred, (pred.shape[0], _D))

    full = lambda shape: pl.BlockSpec(shape, lambda: tuple(0 for _ in shape))
    mpd = mp.shape[1]
    return pl.pallas_call(
        body,
        in_specs=[full((_NREG, _D)), full((_NREG, _D)), full((_NREG, _D)),
                  full((2, mpd)), full((mpd, _D)), full((1, _D)),
                  full((_D, 1)), full((_D, _D)), full((1, _D)),
                  full((_D, 1)), full((1, 1))],
        out_specs=full((_NREG, _D)),
        out_shape=jax.ShapeDtypeStruct((_NREG, _D), _F32),
    )(sem0, sem1, e5k, mp, pw1, pb1, pw2, predw, predb, erw, erb)


def _pad1(x, n, val):
    return jnp.concatenate([x, jnp.full((n - x.shape[0],), val, x.dtype)])


def kernel(edge_index, gs0_edge_index, gs0_eids, gs1_edge_index, gs1_eids,
           metapath_emb, task_desc_emb, E_pretrain, Wm, bm,
           rgcn_Wroot, rgcn_Wrel, rgcn_b, han_Wroot, han_Wrel, han_b,
           projW1, projb1, projW2, predW, predb, erW, erb):
    ep = jnp.concatenate(
        [E_pretrain, jnp.zeros((_NEP - _NE, _D), _F32)], axis=0)
    nef = _NW * _CF * 128
    srcf = _pad1(edge_index[0], nef, 0).reshape(_NW, _CF, 128)
    dstf = _pad1(edge_index[1], nef, _NEP - 1).reshape(_NW, _CF, 128)

    neh = _NS * _CH * 128
    srch = jnp.concatenate([
        _pad1(gs0_edge_index[0], neh, 0),
        _pad1(gs1_edge_index[0] + _NMPP, neh, _NMPP),
    ]).reshape(_NW, _CH, 128)
    dsth = jnp.concatenate([
        _pad1(gs0_edge_index[1], neh, _NMPP - 1),
        _pad1(gs1_edge_index[1], neh, _NMPP - 1),
    ]).reshape(_NW, _CH, 128)

    eids = jnp.concatenate([
        _pad1(gs0_eids, _NMPP, 0), _pad1(gs1_eids, _NMPP, 0),
    ]).reshape(_NW, (2 * _NMPP) // _NW // 128, 128)

    bm2 = bm.reshape(1, _D)
    rb = rgcn_b.reshape(2, 1, _D)
    hb = han_b.reshape(4, 1, _D)

    agg_full = _make_sc_agg(_NEP, _CF, _NEP)
    agg_han = _make_sc_agg(_NMPP, _CH, _NMPP)
    gather = _make_sc_gather(2 * _NMPP, _NEP)

    e0, y0 = _tc_head(ep, Wm, bm2, rgcn_Wrel[0])
    p, c = agg_full(srcf, dstf, y0)
    e1, y1 = _tc_combine_full(p[:_NEP], p[_NEP:], c[:_NEP], c[_NEP:],
                              e0, rgcn_Wroot[0], rb[0], rgcn_Wrel[1])
    p, c = agg_full(srcf, dstf, y1)
    e2 = _tc_combine_full(p[:_NEP], p[_NEP:], c[:_NEP], c[_NEP:],
                          e1, rgcn_Wroot[1], rb[1], None)

    ef = gather(e2, eids)
    wrel_j0 = jnp.stack([han_Wrel[0], han_Wrel[2]])
    wrel_j1 = jnp.stack([han_Wrel[1], han_Wrel[3]])
    wroot_j0 = jnp.stack([han_Wroot[0], han_Wroot[2]])
    wroot_j1 = jnp.stack([han_Wroot[1], han_Wroot[3]])
    b_j0 = jnp.stack([hb[0], hb[2]])
    b_j1 = jnp.stack([hb[1], hb[3]])

    yh = _tc_mm_per_graph(ef, wrel_j0)
    s, c = agg_han(srch, dsth, yh)
    ef1, yh2 = _tc_combine_han(s, c, ef, wroot_j0, b_j0, wrel_j1)
    s, c = agg_han(srch, dsth, yh2)
    ef2 = _tc_combine_han(s, c, ef1, wroot_j1, b_j1, None)

    out = _tc_final(ef2[:_NREG], ef2[_NMPP:_NMPP + _NREG], e2[:_NREG],
                    metapath_emb, projW1, projb1.reshape(1, _D), projW2,
                    predW, predb.reshape(1, _D), erW, erb.reshape(1, 1))
    return out[:, :1]


# speedup vs baseline: 1.5692x; 1.5618x over previous
import jax
import jax.numpy as jnp
from jax import lax
from jax.experimental import pallas as pl
from jax.experimental.pallas import tpu as pltpu
from jax.experimental.pallas import tpu_sc as plsc

_NE, _D = 10000, 128
_NREG = 5000
_NMP = 8000
_EF, _EMP = 320000, 160000
_NC, _NS, _L = 2, 16, 16
_NW = _NC * _NS
_NEP = 10240
_NMPP = 8192
_CF = 79
_CH = 79
_F32 = jnp.float32


def _mesh():
    return plsc.VectorSubcoreMesh(core_axis_name="c", subcore_axis_name="s")


def _fill(ref, n16, val):
    v = jnp.full((_L,), val, _F32)

    def body(i, _):
        ref[pl.ds(i * _L, _L)] = v
        return 0

    lax.fori_loop(0, n16, body, 0)


def _zero_rows(rows):
    z = jnp.zeros((_L,), _F32)

    def body(i, _):
        for j in range(8):
            rows[i, pl.ds(j * _L, _L)] = z
        return 0

    lax.fori_loop(0, 128, body, 0)


def _make_sc_agg(n_rows, chunks, with_counts):
    rpt = n_rows // _NS
    assert rpt % 128 == 0

    def body(src_hbm, dst_hbm, y_hbm, *rest):
        if with_counts:
            (outp, outc, sidx, didx, rows, ones, zc, acc, cacc, sem) = rest
        else:
            (outp, sidx, didx, rows, ones, zc, acc, cacc, sem) = rest
        cid = lax.axis_index("c")
        sid = lax.axis_index("s")
        wid = cid * _NS + sid

        _zero_rows(rows)
        _fill(ones, 128 // _L, 1.0)
        _fill(zc, rpt // _L, 0.0)

        def zacc(k, _):
            pltpu.sync_copy(rows, acc.at[pl.ds(sid * rpt + k * 128, 128)])
            return 0

        lax.fori_loop(0, rpt // 128, zacc, 0)
        pltpu.sync_copy(zc, cacc.at[pl.ds(sid * rpt, rpt)])
        plsc.subcore_barrier()

        pltpu.sync_copy(src_hbm.at[wid], sidx)
        pltpu.sync_copy(dst_hbm.at[wid], didx)

        def step(j, _):
            pltpu.async_copy(y_hbm.at[sidx.at[j]], rows, sem).wait()
            pltpu.sync_copy(rows, acc.at[didx.at[j]], add=True)
            if with_counts:
                pltpu.sync_copy(ones, cacc.at[didx.at[j]], add=True)
            return 0

        lax.fori_loop(0, chunks, step, 0)
        plsc.subcore_barrier()

        def wout(k, _):
            r = sid * rpt + k * 128
            pltpu.sync_copy(acc.at[pl.ds(r, 128)],
                            outp.at[pl.ds(cid * n_rows + r, 128)])
            return 0

        lax.fori_loop(0, rpt // 128, wout, 0)
        if with_counts:
            pltpu.sync_copy(cacc.at[pl.ds(sid * rpt, rpt)],
                            outc.at[pl.ds(cid * n_rows + sid * rpt, rpt)])

    out_type = [jax.ShapeDtypeStruct((_NC * n_rows, _D), _F32)]
    if with_counts:
        out_type.append(jax.ShapeDtypeStruct((_NC * n_rows,), _F32))
    scratch = [
        pltpu.VMEM((chunks, 128), jnp.int32),
        pltpu.VMEM((chunks, 128), jnp.int32),
        pltpu.VMEM((128, _D), _F32),
        pltpu.VMEM((128,), _F32),
        pltpu.VMEM((rpt,), _F32),
    ]
    scratch += [
        pltpu.VMEM_SHARED((n_rows, _D), _F32),
        pltpu.VMEM_SHARED((n_rows,), _F32),
        pltpu.SemaphoreType.DMA,
    ]
    return pl.kernel(
        body,
        out_type=out_type if with_counts else out_type[0],
        mesh=_mesh(),
        scratch_types=scratch,
    )


def _make_sc_gather(n_out, table_rows):
    kpw = n_out // _NW // 128

    def body(table_hbm, idx_hbm, out_hbm, idxv, rows, sem):
        cid = lax.axis_index("c")
        sid = lax.axis_index("s")
        wid = cid * _NS + sid
        pltpu.sync_copy(idx_hbm.at[wid], idxv)
        for k in range(kpw):
            pltpu.async_copy(table_hbm.at[idxv.at[k]], rows, sem).wait()
            pltpu.sync_copy(rows, out_hbm.at[pl.ds(wid * kpw * 128 + k * 128, 128)])

    return pl.kernel(
        body,
        out_type=jax.ShapeDtypeStruct((n_out, _D), _F32),
        mesh=_mesh(),
        scratch_types=[
            pltpu.VMEM((kpw, 128), jnp.int32),
            pltpu.VMEM((128, _D), _F32),
            pltpu.SemaphoreType.DMA,
        ],
    )


_BLK = 1024


def _row_spec(blk=_BLK):
    return pl.BlockSpec((blk, _D), lambda i: (i, 0))


def _w_spec():
    return pl.BlockSpec((_D, _D), lambda i: (0, 0))


def _b_spec():
    return pl.BlockSpec((1, _D), lambda i: (0, 0))


def _tc_head(ep, wm, bm, wrel):
    def body(ep_r, wm_r, bm_r, wr_r, e_out, y_out):
        e = jnp.dot(ep_r[...], wm_r[...], preferred_element_type=_F32) + bm_r[...]
        e_out[...] = e
        y_out[...] = jnp.dot(e, wr_r[...], preferred_element_type=_F32)

    n = ep.shape[0]
    return pl.pallas_call(
        body,
        grid=(n // _BLK,),
        in_specs=[_row_spec(), _w_spec(), _b_spec(), _w_spec()],
        out_specs=[_row_spec(), _row_spec()],
        out_shape=[jax.ShapeDtypeStruct((n, _D), _F32)] * 2,
    )(ep, wm, bm, wrel)


def _tc_combine_full(p0, p1, c0, c1, e, wroot, b, wrel_next):
    with_next = wrel_next is not None

    def body(p0_r, p1_r, c0_r, c1_r, e_r, wr_r, b_r, *rest):
        if with_next:
            wn_r, e_out, y_out = rest
        else:
            (e_out,) = rest
        cnt = jnp.maximum(c0_r[...] + c1_r[...], 1.0)
        agg = (p0_r[...] + p1_r[...]) / cnt[:, None]
        enew = jnp.tanh(agg + jnp.dot(e_r[...], wr_r[...],
                                      preferred_element_type=_F32) + b_r[...])
        e_out[...] = enew
        if with_next:
            y_out[...] = jnp.dot(enew, wn_r[...], preferred_element_type=_F32)

    n = e.shape[0]
    c_spec = pl.BlockSpec((_BLK,), lambda i: (i,))
    in_specs = [_row_spec(), _row_spec(), c_spec, c_spec, _row_spec(),
                _w_spec(), _b_spec()]
    args = [p0, p1, c0, c1, e, wroot, b]
    out_specs = [_row_spec()]
    out_shape = [jax.ShapeDtypeStruct((n, _D), _F32)]
    if with_next:
        in_specs.append(_w_spec())
        args.append(wrel_next)
        out_specs.append(_row_spec())
        out_shape.append(jax.ShapeDtypeStruct((n, _D), _F32))
    res = pl.pallas_call(
        body, grid=(n // _BLK,), in_specs=in_specs, out_specs=out_specs,
        out_shape=out_shape,
    )(*args)
    return res if with_next else res[0]


def _tc_mm_per_graph(x, w):
    gblk = _NMPP // _BLK

    def body(x_r, w_r, y_out):
        y_out[...] = jnp.dot(x_r[...], w_r[0], preferred_element_type=_F32)

    n = x.shape[0]
    return pl.pallas_call(
        body,
        grid=(n // _BLK,),
        in_specs=[_row_spec(),
                  pl.BlockSpec((1, _D, _D), lambda i: (i // gblk, 0, 0))],
        out_specs=_row_spec(),
        out_shape=jax.ShapeDtypeStruct((n, _D), _F32),
    )(x, w)


def _tc_combine_han(s, c, e, wroot, b, wrel_next):
    with_next = wrel_next is not None
    gblk = _NMPP // _BLK

    def body(s_r, c_r, e_r, wr_r, b_r, *rest):
        if with_next:
            wn_r, e_out, y_out = rest
        else:
            (e_out,) = rest
        cnt = jnp.maximum(c_r[...], 1.0)
        agg = s_r[...] / cnt[:, None]
        enew = jax.nn.relu(agg + jnp.dot(e_r[...], wr_r[0],
                                         preferred_element_type=_F32) + b_r[0])
        e_out[...] = enew
        if with_next:
            y_out[...] = jnp.dot(enew, wn_r[0], preferred_element_type=_F32)

    n = e.shape[0]
    wg_spec = pl.BlockSpec((1, _D, _D), lambda i: (i // gblk, 0, 0))
    bg_spec = pl.BlockSpec((1, 1, _D), lambda i: (i // gblk, 0, 0))
    c_spec = pl.BlockSpec((_BLK,), lambda i: (i,))
    in_specs = [_row_spec(), c_spec, _row_spec(), wg_spec, bg_spec]
    args = [s, c, e, wroot, b]
    out_specs = [_row_spec()]
    out_shape = [jax.ShapeDtypeStruct((n, _D), _F32)]
    if with_next:
        in_specs.append(wg_spec)
        args.append(wrel_next)
        out_specs.append(_row_spec())
        out_shape.append(jax.ShapeDtypeStruct((n, _D), _F32))
    res = pl.pallas_call(
        body, grid=(n // _BLK,), in_specs=in_specs, out_specs=out_specs,
        out_shape=out_shape,
    )(*args)
    return res if with_next else res[0]


def _tc_final(sem0, sem1, e5k, mp, pw1, pb1, pw2, predw, predb, erw, erb):
    def body(s0_r, s1_r, e_r, mp_r, pw1_r, pb1_r, pw2_r, prw_r, prb_r, erw_r,
             erb_r, out):
        t = jnp.tanh(jnp.dot(mp_r[...], pw1_r[...],
                             preferred_element_type=_F32) + pb1_r[...])
        w = jnp.dot(t, pw2_r[...], preferred_element_type=_F32)
        m = jnp.max(w, axis=0, keepdims=True)
        ew = jnp.exp(w - m)
        wn = ew / jnp.sum(ew, axis=0, keepdims=True)
        h = s0_r[...] * wn[0:1, :] + s1_r[...] * wn[1:2, :]
        ereg = jnp.dot(h, prw_r[...], preferred_element_type=_F32) + prb_r[...]
        ereg = ereg + e_r[...]
        pred = jnp.dot(ereg, erw_r[...], preferred_element_type=_F32) + erb_r[...]
        out[...] = jnp.broadcast_to(pred, (pred.shape[0], _D))

    full = lambda shape: pl.BlockSpec(shape, lambda: tuple(0 for _ in shape))
    mpd = mp.shape[1]
    return pl.pallas_call(
        body,
        in_specs=[full((_NREG, _D)), full((_NREG, _D)), full((_NREG, _D)),
                  full((2, mpd)), full((mpd, _D)), full((1, _D)),
                  full((_D, 1)), full((_D, _D)), full((1, _D)),
                  full((_D, 1)), full((1, 1))],
        out_specs=full((_NREG, _D)),
        out_shape=jax.ShapeDtypeStruct((_NREG, _D), _F32),
    )(sem0, sem1, e5k, mp, pw1, pb1, pw2, predw, predb, erw, erb)


def _pad1(x, n, val):
    return jnp.concatenate([x, jnp.full((n - x.shape[0],), val, x.dtype)])


def kernel(edge_index, gs0_edge_index, gs0_eids, gs1_edge_index, gs1_eids,
           metapath_emb, task_desc_emb, E_pretrain, Wm, bm,
           rgcn_Wroot, rgcn_Wrel, rgcn_b, han_Wroot, han_Wrel, han_b,
           projW1, projb1, projW2, predW, predb, erW, erb):
    ep = jnp.concatenate(
        [E_pretrain, jnp.zeros((_NEP - _NE, _D), _F32)], axis=0)
    nef = _NW * _CF * 128
    srcf = _pad1(edge_index[0], nef, 0).reshape(_NW, _CF, 128)
    dstf = _pad1(edge_index[1], nef, _NEP - 1).reshape(_NW, _CF, 128)

    neh = _NS * _CH * 128
    srch = jnp.concatenate([
        _pad1(gs0_edge_index[0], neh, 0),
        _pad1(gs1_edge_index[0] + _NMPP, neh, _NMPP),
    ]).reshape(_NW, _CH, 128)
    dsth = jnp.concatenate([
        _pad1(gs0_edge_index[1], neh, _NMPP - 1),
        _pad1(gs1_edge_index[1], neh, _NMPP - 1),
    ]).reshape(_NW, _CH, 128)

    eids = jnp.concatenate([
        _pad1(gs0_eids, _NMPP, 0), _pad1(gs1_eids, _NMPP, 0),
    ]).reshape(_NW, (2 * _NMPP) // _NW // 128, 128)

    bm2 = bm.reshape(1, _D)
    rb = rgcn_b.reshape(2, 1, _D)
    hb = han_b.reshape(4, 1, _D)

    agg_full_c = _make_sc_agg(_NEP, _CF, True)
    agg_full_n = _make_sc_agg(_NEP, _CF, False)
    agg_han_c = _make_sc_agg(_NMPP, _CH, True)
    agg_han_n = _make_sc_agg(_NMPP, _CH, False)
    gather = _make_sc_gather(2 * _NMPP, _NEP)

    e0, y0 = _tc_head(ep, Wm, bm2, rgcn_Wrel[0])
    p, c = agg_full_c(srcf, dstf, y0)
    e1, y1 = _tc_combine_full(p[:_NEP], p[_NEP:], c[:_NEP], c[_NEP:],
                              e0, rgcn_Wroot[0], rb[0], rgcn_Wrel[1])
    p = agg_full_n(srcf, dstf, y1)
    e2 = _tc_combine_full(p[:_NEP], p[_NEP:], c[:_NEP], c[_NEP:],
                          e1, rgcn_Wroot[1], rb[1], None)

    ef = gather(e2, eids)
    wrel_j0 = jnp.stack([han_Wrel[0], han_Wrel[2]])
    wrel_j1 = jnp.stack([han_Wrel[1], han_Wrel[3]])
    wroot_j0 = jnp.stack([han_Wroot[0], han_Wroot[2]])
    wroot_j1 = jnp.stack([han_Wroot[1], han_Wroot[3]])
    b_j0 = jnp.stack([hb[0], hb[2]])
    b_j1 = jnp.stack([hb[1], hb[3]])

    yh = _tc_mm_per_graph(ef, wrel_j0)
    s, c = agg_han_c(srch, dsth, yh)
    ef1, yh2 = _tc_combine_han(s, c, ef, wroot_j0, b_j0, wrel_j1)
    s = agg_han_n(srch, dsth, yh2)
    ef2 = _tc_combine_han(s, c, ef1, wroot_j1, b_j1, None)

    out = _tc_final(ef2[:_NREG], ef2[_NMPP:_NMPP + _NREG], e2[:_NREG],
                    metapath_emb, projW1, projb1.reshape(1, _D), projW2,
                    predW, predb.reshape(1, _D), erW, erb.reshape(1, 1))
    return out[:, :1]
